# baseline (speedup 1.0000x reference)
import jax
import jax.numpy as jnp
from jax import lax
from jax.experimental import pallas as pl
from jax.experimental.pallas import tpu as pltpu
from jax.experimental.pallas import tpu_sc as plsc

VOCAB = 1000
VOCAB_PAD = 1024
BATCH = 1024
SEQ = 50

NC = 2
NS = 16
NW = NC * NS

B = BATCH * SEQ
B_PER_W = B // NW
CHUNK = 16
N_CHUNKS = B_PER_W // CHUNK

LANES = 16
N_FULL_VECS = VOCAB // LANES
TAIL_OFF = VOCAB - LANES


def _narrow_rows(src, dst):
    @pl.loop(0, CHUNK)
    def _(r):
        for j in range(N_FULL_VECS):
            slc = (pl.ds(r, 1), pl.ds(j * LANES, LANES))
            dst.at[slc[0], slc[1]][...] = src.at[slc[0], slc[1]][...]
        tail = (pl.ds(r, 1), pl.ds(TAIL_OFF, LANES))
        dst.at[tail[0], tail[1]][...] = src.at[tail[0], tail[1]][...]


def _gather_kernel(
    table_hbm, idx_hbm, out_hbm,
    idx_v, gbuf0, gbuf1, sbuf0, sbuf1, gsem0, gsem1, ssem0, ssem1
):
    cid = lax.axis_index("c")
    sid = lax.axis_index("s")
    wid = sid * NC + cid
    base = wid * B_PER_W

    pltpu.sync_copy(idx_hbm.at[pl.ds(base, B_PER_W)], idx_v)

    bufs = ((gbuf0, sbuf0, gsem0, ssem0), (gbuf1, sbuf1, gsem1, ssem1))

    for b in range(2):
        gbuf, _, gsem, _ = bufs[b]
        pltpu.make_async_copy(
            table_hbm.at[idx_v.at[pl.ds(b * CHUNK, CHUNK)]], gbuf, gsem
        ).start()

    @pl.loop(0, N_CHUNKS // 2)
    def _(p):
        for b in range(2):
            gbuf, sbuf, gsem, ssem = bufs[b]
            c = p * 2 + b
            off = c * CHUNK

            pltpu.make_async_copy(
                table_hbm.at[idx_v.at[pl.ds(off, CHUNK)]], gbuf, gsem
            ).wait()

            @pl.when(c >= 2)
            def _():
                pltpu.make_async_copy(
                    sbuf, out_hbm.at[pl.ds(base + (c - 2) * CHUNK, CHUNK)], ssem
                ).wait()

            _narrow_rows(gbuf, sbuf)

            pltpu.make_async_copy(
                sbuf, out_hbm.at[pl.ds(base + off, CHUNK)], ssem
            ).start()

            @pl.when(c + 2 < N_CHUNKS)
            def _():
                pltpu.make_async_copy(
                    table_hbm.at[idx_v.at[pl.ds(off + 2 * CHUNK, CHUNK)]],
                    gbuf, gsem,
                ).start()

    for b in range(2):
        _, sbuf, _, ssem = bufs[b]
        c = N_CHUNKS - 2 + b
        pltpu.make_async_copy(
            sbuf, out_hbm.at[pl.ds(base + c * CHUNK, CHUNK)], ssem
        ).wait()


@jax.jit
def _gather(table_pad, idx_flat):
    mesh = plsc.VectorSubcoreMesh(core_axis_name="c", subcore_axis_name="s")
    k = pl.kernel(
        _gather_kernel,
        out_type=jax.ShapeDtypeStruct((B, VOCAB), jnp.float32),
        mesh=mesh,
        scratch_types=[
            pltpu.VMEM((B_PER_W,), jnp.int32),
            pltpu.VMEM((CHUNK, VOCAB_PAD), jnp.float32),
            pltpu.VMEM((CHUNK, VOCAB_PAD), jnp.float32),
            pltpu.VMEM((CHUNK, VOCAB), jnp.float32),
            pltpu.VMEM((CHUNK, VOCAB), jnp.float32),
            pltpu.SemaphoreType.DMA,
            pltpu.SemaphoreType.DMA,
            pltpu.SemaphoreType.DMA,
            pltpu.SemaphoreType.DMA,
        ],
    )
    return k(table_pad, idx_flat)


def kernel(table, idx):
    table_pad = jnp.pad(table, ((0, 0), (0, VOCAB_PAD - VOCAB)))
    out = _gather(table_pad, idx.reshape(-1))
    return out.reshape(BATCH, SEQ, VOCAB)


# speedup vs baseline: 1.0157x; 1.0157x over previous
import jax
import jax.numpy as jnp
from jax import lax
from jax.experimental import pallas as pl
from jax.experimental.pallas import tpu as pltpu
from jax.experimental.pallas import tpu_sc as plsc

VOCAB = 1000
VOCAB_PAD = 1024
BATCH = 1024
SEQ = 50

NC = 2
NS = 16
NW = NC * NS

B = BATCH * SEQ
B_PER_W = B // NW
CHUNK = 40
N_CHUNKS = B_PER_W // CHUNK

HEAD = 896
TILE = 128


def _gather_kernel(
    table_hbm, idx_hbm, out_hbm, idx_v, gbuf0, gbuf1, gsem0, gsem1, ssem0, ssem1
):
    cid = lax.axis_index("c")
    sid = lax.axis_index("s")
    wid = sid * NC + cid
    base = wid * B_PER_W

    pltpu.sync_copy(idx_hbm.at[pl.ds(base, B_PER_W)], idx_v)

    tail_off = pl.multiple_of(HEAD + 0 * wid, TILE)

    def store(gbuf, ssem, off):
        dst = out_hbm.at[pl.ds(base + off, CHUNK)]
        pltpu.make_async_copy(
            gbuf.at[:, pl.ds(0, HEAD)], dst.at[:, pl.ds(0, HEAD)], ssem
        ).start()
        pltpu.make_async_copy(
            gbuf.at[:, pl.ds(HEAD, TILE)], dst.at[:, pl.ds(tail_off, TILE)], ssem
        ).start()

    def store_wait(gbuf, ssem, off):
        dst = out_hbm.at[pl.ds(base + off, CHUNK)]
        pltpu.make_async_copy(
            gbuf.at[:, pl.ds(0, HEAD)], dst.at[:, pl.ds(0, HEAD)], ssem
        ).wait()
        pltpu.make_async_copy(
            gbuf.at[:, pl.ds(HEAD, TILE)], dst.at[:, pl.ds(tail_off, TILE)], ssem
        ).wait()

    bufs = ((gbuf0, gsem0, ssem0), (gbuf1, gsem1, ssem1))

    for b in range(2):
        gbuf, gsem, _ = bufs[b]
        pltpu.make_async_copy(
            table_hbm.at[idx_v.at[pl.ds(b * CHUNK, CHUNK)]], gbuf, gsem
        ).start()

    @pl.loop(0, N_CHUNKS // 2)
    def _(p):
        for b in range(2):
            gbuf, gsem, ssem = bufs[b]
            c = p * 2 + b
            off = c * CHUNK

            pltpu.make_async_copy(
                table_hbm.at[idx_v.at[pl.ds(off, CHUNK)]], gbuf, gsem
            ).wait()

            store(gbuf, ssem, off)

            @pl.when(c + 2 < N_CHUNKS)
            def _():
                store_wait(gbuf, ssem, off)
                pltpu.make_async_copy(
                    table_hbm.at[idx_v.at[pl.ds(off + 2 * CHUNK, CHUNK)]],
                    gbuf, gsem,
                ).start()

    for b in range(2):
        gbuf, _, ssem = bufs[b]
        c = N_CHUNKS - 2 + b
        store_wait(gbuf, ssem, c * CHUNK)


@jax.jit
def _gather(table_pad, idx_flat):
    mesh = plsc.VectorSubcoreMesh(core_axis_name="c", subcore_axis_name="s")
    k = pl.kernel(
        _gather_kernel,
        out_type=jax.ShapeDtypeStruct((B, VOCAB), jnp.float32),
        mesh=mesh,
        compiler_params=pltpu.CompilerParams(disable_bounds_checks=True),
        scratch_types=[
            pltpu.VMEM((B_PER_W,), jnp.int32),
            pltpu.VMEM((CHUNK, VOCAB_PAD), jnp.float32),
            pltpu.VMEM((CHUNK, VOCAB_PAD), jnp.float32),
            pltpu.SemaphoreType.DMA,
            pltpu.SemaphoreType.DMA,
            pltpu.SemaphoreType.DMA,
            pltpu.SemaphoreType.DMA,
        ],
    )
    return k(table_pad, idx_flat)


def kernel(table, idx):
    table_pad = jnp.pad(table, ((0, 0), (0, VOCAB_PAD - VOCAB)))
    out = _gather(table_pad, idx.reshape(-1))
    return out.reshape(BATCH, SEQ, VOCAB)
